# baseline (speedup 1.0000x reference)
import functools

import jax
import jax.numpy as jnp
from jax.experimental import pallas as pl
from jax.experimental.pallas import tpu as pltpu

_RB = 32


def _loss_kernel(lab_ref, clab_ref, s_ref, out_ref, acc_ref, *, total):
    i = pl.program_id(0)
    nt = pl.num_programs(0)

    @pl.when(i == 0)
    def _init():
        acc_ref[0] = 0.0
        acc_ref[1] = 0.0
        acc_ref[2] = 0.0

    lab = lab_ref[:]
    clab = clab_ref[:]
    s = s_ref[:]
    m = lab == clab
    t2 = jnp.maximum(s, 0.0)
    t1 = jnp.maximum(1.0 - s, 0.0)
    acc_ref[0] += jnp.sum(jnp.where(m, 0.0, t2))
    acc_ref[1] += jnp.sum(jnp.where(m, t1, 0.0))
    acc_ref[2] += jnp.sum(m.astype(jnp.float32))

    @pl.when(i == nt - 1)
    def _fin():
        n_match = acc_ref[2]
        n_non = jnp.float32(total) - n_match
        out_ref[0] = acc_ref[1] / n_match + acc_ref[0] / n_non


def kernel(fuse_scores, labels, center_labels):
    with jax.enable_x64(False):
        return _run(fuse_scores, labels, center_labels)


def _run(fuse_scores, labels, center_labels):
    B, N = fuse_scores.shape
    nt = B // _RB
    lab2d = labels.astype(jnp.int32).reshape(B, 1)
    clab2d = center_labels.astype(jnp.int32).reshape(1, N)

    out = pl.pallas_call(
        functools.partial(_loss_kernel, total=float(B) * float(N)),
        grid=(nt,),
        in_specs=[
            pl.BlockSpec((_RB, 1), lambda i: (i, 0)),
            pl.BlockSpec((1, N), lambda i: (0, 0)),
            pl.BlockSpec((_RB, N), lambda i: (i, 0)),
        ],
        out_specs=pl.BlockSpec(memory_space=pltpu.SMEM),
        out_shape=jax.ShapeDtypeStruct((1,), jnp.float32),
        scratch_shapes=[
            pltpu.SMEM((3,), jnp.float32),
        ],
        compiler_params=pltpu.CompilerParams(
            vmem_limit_bytes=128 * 1024 * 1024,
        ),
    )(lab2d, clab2d, fuse_scores)
    score = out[0]
    return (score, score)
